# baseline (speedup 1.0000x reference)
import functools

import jax
import jax.numpy as jnp
from jax import lax
from jax.experimental import pallas as pl
from jax.experimental.pallas import tpu as pltpu
from jax.experimental.pallas import tpu_sc as plsc

B = 16384
K = 50
NUM_TYPES = 27
GLOVE_DIM = 300
EMBED_DIM = 16

NW = 32
CHUNK = 800
N_TOTAL = B * K
PER_W = N_TOTAL // NW
N_CHUNKS = PER_W // CHUNK


def _table_body(glove_ref, w_ref, b_ref, table_ref):
    g = glove_ref[...]
    w = w_ref[...]
    t = lax.dot_general(g, w, (((1,), (1,)), ((), ())),
                        preferred_element_type=jnp.float32)
    table_ref[...] = t + b_ref[...]


def _project_table(glove, W, b):
    return pl.pallas_call(
        _table_body,
        out_shape=jax.ShapeDtypeStruct((NUM_TYPES, EMBED_DIM), jnp.float32),
    )(glove, W, b.reshape(1, EMBED_DIM))


def _sc_gather_body(table_hbm, idx_hbm, out_hbm,
                    idx_v, rows0, rows1, g0, g1, s0, s1):
    wid = lax.axis_index("s") * 2 + lax.axis_index("c")
    pltpu.sync_copy(idx_hbm.at[wid], idx_v)

    rows = (rows0, rows1)
    gsem = (g0, g1)
    ssem = (s0, s1)

    def fire(step, buf):
        pltpu.async_copy(table_hbm.at[idx_v.at[step]], rows[buf], gsem[buf])

    def drain(step, buf):
        pltpu.make_async_copy(table_hbm.at[idx_v.at[step]], rows[buf],
                              gsem[buf]).wait()

    def store(step, buf):
        pltpu.async_copy(rows[buf], out_hbm.at[wid, step], ssem[buf])

    def store_wait(step, buf):
        pltpu.make_async_copy(rows[buf], out_hbm.at[wid, step],
                              ssem[buf]).wait()

    fire(0, 0)
    drain(0, 0)
    store(0, 0)
    fire(1, 1)

    @pl.loop(1, N_CHUNKS - 2, step=2)
    def _pair(t):
        drain(t, 1)
        store_wait(t - 1, 0)
        fire(t + 1, 0)
        store(t, 1)
        drain(t + 1, 0)
        store_wait(t, 1)
        fire(t + 2, 1)
        store(t + 1, 0)

    t_last = N_CHUNKS - 1
    drain(t_last, 1)
    store_wait(t_last - 1, 0)
    store(t_last, 1)
    store_wait(t_last, 1)


_sc_gather = functools.partial(
    pl.kernel,
    out_type=jax.ShapeDtypeStruct((NW, N_CHUNKS, CHUNK, EMBED_DIM),
                                  jnp.float32),
    mesh=plsc.VectorSubcoreMesh(core_axis_name="c", subcore_axis_name="s"),
    scratch_types=[
        pltpu.VMEM((N_CHUNKS, CHUNK), jnp.int32),
        pltpu.VMEM((CHUNK, EMBED_DIM), jnp.float32),
        pltpu.VMEM((CHUNK, EMBED_DIM), jnp.float32),
        pltpu.SemaphoreType.DMA,
        pltpu.SemaphoreType.DMA,
        pltpu.SemaphoreType.DMA,
        pltpu.SemaphoreType.DMA,
    ],
    compiler_params=pltpu.CompilerParams(use_tc_tiling_on_sc=False),
)(_sc_gather_body)


@jax.jit
def kernel(type_indices, glove_embeddings, W, b):
    table = _project_table(glove_embeddings, W, b)
    idx = type_indices.reshape(NW, N_CHUNKS, CHUNK).astype(jnp.int32)
    out = _sc_gather(table, idx)
    return out.reshape(B, K, EMBED_DIM)


# speedup vs baseline: 2.4652x; 2.4652x over previous
import functools

import jax
import jax.numpy as jnp
from jax import lax
from jax.experimental import pallas as pl
from jax.experimental.pallas import tpu as pltpu
from jax.experimental.pallas import tpu_sc as plsc

B = 16384
K = 50
NUM_TYPES = 27
GLOVE_DIM = 300
EMBED_DIM = 16

NW = 32
CHUNK = 800
N_TOTAL = B * K
PER_W = N_TOTAL // NW
N_CHUNKS = PER_W // CHUNK


def _table_body(glove_ref, w_ref, b_ref, table_ref):
    g = glove_ref[...]
    w = w_ref[...]
    t = lax.dot_general(g, w, (((1,), (1,)), ((), ())),
                        preferred_element_type=jnp.float32)
    table_ref[...] = t + b_ref[...]


def _project_table(glove, W, b):
    return pl.pallas_call(
        _table_body,
        out_shape=jax.ShapeDtypeStruct((NUM_TYPES, EMBED_DIM), jnp.float32),
    )(glove, W, b.reshape(1, EMBED_DIM))


BLOCKS = CHUNK // 16
CHUNK_F = CHUNK * EMBED_DIM


def _sc_gather_body(table_hbm, idx_hbm, out_hbm,
                    table_v, idx_v, out0, out1, s0, s1):
    wid = lax.axis_index("s") * 2 + lax.axis_index("c")
    pltpu.sync_copy(table_hbm, table_v)
    pltpu.sync_copy(idx_hbm.at[wid], idx_v)

    outbuf = (out0, out1)
    ssem = (s0, s1)

    def compute(c, buf):
        ob = outbuf[buf]

        @pl.loop(0, BLOCKS)
        def _blk(bi):
            idxv = idx_v[pl.ds(c * CHUNK + bi * 16, 16)]
            base = idxv * EMBED_DIM
            pos = lax.iota(jnp.int32, 16) * EMBED_DIM + bi * (16 * EMBED_DIM)
            for j in range(EMBED_DIM):
                col = plsc.load_gather(table_v, [base + j])
                plsc.store_scatter(ob, [pos + j], col)

    def store(c, buf):
        pltpu.async_copy(outbuf[buf], out_hbm.at[wid, c], ssem[buf])

    def store_wait(c, buf):
        pltpu.make_async_copy(outbuf[buf], out_hbm.at[wid, c],
                              ssem[buf]).wait()

    compute(0, 0)
    store(0, 0)
    compute(1, 1)
    store(1, 1)

    @pl.loop(2, N_CHUNKS, step=2)
    def _pair(c):
        store_wait(c - 2, 0)
        compute(c, 0)
        store(c, 0)
        store_wait(c - 1, 1)
        compute(c + 1, 1)
        store(c + 1, 1)

    store_wait(N_CHUNKS - 2, 0)
    store_wait(N_CHUNKS - 1, 1)


_sc_gather = functools.partial(
    pl.kernel,
    out_type=jax.ShapeDtypeStruct((NW, N_CHUNKS, CHUNK_F), jnp.float32),
    mesh=plsc.VectorSubcoreMesh(core_axis_name="c", subcore_axis_name="s"),
    scratch_types=[
        pltpu.VMEM((NUM_TYPES * EMBED_DIM,), jnp.float32),
        pltpu.VMEM((PER_W,), jnp.int32),
        pltpu.VMEM((CHUNK_F,), jnp.float32),
        pltpu.VMEM((CHUNK_F,), jnp.float32),
        pltpu.SemaphoreType.DMA,
        pltpu.SemaphoreType.DMA,
    ],
    compiler_params=pltpu.CompilerParams(use_tc_tiling_on_sc=False,
                                         needs_layout_passes=False),
)(_sc_gather_body)


@jax.jit
def kernel(type_indices, glove_embeddings, W, b):
    table = _project_table(glove_embeddings, W, b).reshape(-1)
    idx = type_indices.reshape(NW, PER_W).astype(jnp.int32)
    out = _sc_gather(table, idx)
    return out.reshape(B, K, EMBED_DIM)


# speedup vs baseline: 12.0161x; 4.8743x over previous
import functools

import jax
import jax.numpy as jnp
from jax import lax
from jax.experimental import pallas as pl
from jax.experimental.pallas import tpu as pltpu
from jax.experimental.pallas import tpu_sc as plsc

B = 16384
K = 50
NUM_TYPES = 27
GLOVE_DIM = 300
EMBED_DIM = 16
TPAD = 32

NW = 32
BT_PER_W = 4
B_PER_W = BT_PER_W * 128
N_BBLK = B_PER_W // 16


def _table_body(glove_ref, w_ref, b_ref, table_ref):
    g = glove_ref[...]
    w = w_ref[...]
    t = lax.dot_general(w, g, (((1,), (1,)), ((), ())),
                        preferred_element_type=jnp.float32)
    table_ref[:, :NUM_TYPES] = t + b_ref[...]


def _project_table(glove, W, b):
    return pl.pallas_call(
        _table_body,
        out_shape=jax.ShapeDtypeStruct((EMBED_DIM, TPAD), jnp.float32),
    )(glove, W, b.reshape(EMBED_DIM, 1))


def _sc_gather_body(tt_hbm, idx_hbm, out_hbm, *scratch):
    te = scratch[:EMBED_DIM]
    idx_v, st0, st1, s0, s1 = scratch[EMBED_DIM:]
    wid = lax.axis_index("s") * 2 + lax.axis_index("c")

    for e in range(EMBED_DIM):
        pltpu.sync_copy(tt_hbm.at[e], te[e])
    pltpu.sync_copy(idx_hbm.at[:, pl.ds(wid * B_PER_W, B_PER_W)], idx_v)

    staging = (st0, st1)
    ssem = (s0, s1)

    def compute(k, buf):
        st = staging[buf]

        @pl.loop(0, N_BBLK)
        def _blk(bb):
            btl = bb // 8
            cs = (bb % 8) * 16
            idxv = idx_v[k, pl.ds(bb * 16, 16)]
            for e in range(EMBED_DIM):
                col = plsc.load_gather(te[e], [idxv])
                st[e // 8, btl, e % 8, pl.ds(cs, 16)] = col

    def store(k, buf):
        st = staging[buf]
        pltpu.async_copy(st.at[0], out_hbm.at[k, 0, pl.ds(wid * BT_PER_W,
                                                          BT_PER_W)],
                         ssem[buf])
        pltpu.async_copy(st.at[1], out_hbm.at[k, 1, pl.ds(wid * BT_PER_W,
                                                          BT_PER_W)],
                         ssem[buf])

    def store_wait(k, buf):
        st = staging[buf]
        for et in range(2):
            pltpu.make_async_copy(
                st.at[et],
                out_hbm.at[k, et, pl.ds(wid * BT_PER_W, BT_PER_W)],
                ssem[buf]).wait()

    compute(0, 0)
    store(0, 0)
    compute(1, 1)
    store(1, 1)

    @pl.loop(2, K, step=2)
    def _pair(k):
        store_wait(k - 2, 0)
        compute(k, 0)
        store(k, 0)
        store_wait(k - 1, 1)
        compute(k + 1, 1)
        store(k + 1, 1)

    store_wait(K - 2, 0)
    store_wait(K - 1, 1)


_sc_gather = functools.partial(
    pl.kernel,
    out_type=jax.ShapeDtypeStruct((K, 2, 128, 8, 128), jnp.float32),
    mesh=plsc.VectorSubcoreMesh(core_axis_name="c", subcore_axis_name="s"),
    scratch_types=(
        [pltpu.VMEM((TPAD,), jnp.float32) for _ in range(EMBED_DIM)]
        + [
            pltpu.VMEM((K, B_PER_W), jnp.int32),
            pltpu.VMEM((2, BT_PER_W, 8, 128), jnp.float32),
            pltpu.VMEM((2, BT_PER_W, 8, 128), jnp.float32),
            pltpu.SemaphoreType.DMA,
            pltpu.SemaphoreType.DMA,
        ]
    ),
    compiler_params=pltpu.CompilerParams(use_tc_tiling_on_sc=False,
                                         needs_layout_passes=False),
)(_sc_gather_body)


@jax.jit
def kernel(type_indices, glove_embeddings, W, b):
    table_t = _project_table(glove_embeddings, W, b)
    idx_t = type_indices.T.astype(jnp.int32)
    out5 = _sc_gather(table_t, idx_t)
    return out5.transpose(2, 4, 0, 1, 3).reshape(B, K, EMBED_DIM)


# speedup vs baseline: 13.2596x; 1.1035x over previous
import functools

import jax
import jax.numpy as jnp
from jax import lax
from jax.experimental import pallas as pl
from jax.experimental.pallas import tpu as pltpu
from jax.experimental.pallas import tpu_sc as plsc

B = 16384
K = 50
NUM_TYPES = 27
GLOVE_DIM = 300
EMBED_DIM = 16
TPAD = 32

NW = 32
BT_PER_W = 4
B_PER_W = BT_PER_W * 128
N_BBLK = B_PER_W // 16


def _table_body(glove_ref, w_ref, b_ref, table_ref):
    g = glove_ref[...]
    w = w_ref[...]
    t = lax.dot_general(w, g, (((1,), (1,)), ((), ())),
                        preferred_element_type=jnp.float32)
    table_ref[:, :NUM_TYPES] = t + b_ref[...]


def _project_table(glove, W, b):
    return pl.pallas_call(
        _table_body,
        out_shape=jax.ShapeDtypeStruct((EMBED_DIM, TPAD), jnp.float32),
    )(glove, W, b.reshape(EMBED_DIM, 1))


def _sc_gather_body(tt_hbm, idx_hbm, out_hbm, *scratch):
    te = scratch[:EMBED_DIM]
    idx_v, st0, st1, s0, s1 = scratch[EMBED_DIM:]
    wid = lax.axis_index("s") * 2 + lax.axis_index("c")

    for e in range(EMBED_DIM):
        pltpu.sync_copy(tt_hbm.at[e], te[e])
    pltpu.sync_copy(idx_hbm.at[:, pl.ds(wid * B_PER_W, B_PER_W)], idx_v)

    staging = (st0, st1)
    ssem = (s0, s1)

    def compute(k, buf):
        st = staging[buf]

        @pl.loop(0, N_BBLK, unroll=4)
        def _blk(bb):
            btl = bb // 8
            cs = (bb % 8) * 16
            idxv = idx_v[k, pl.ds(bb * 16, 16)]
            for e in range(EMBED_DIM):
                col = plsc.load_gather(te[e], [idxv])
                st[e // 8, btl, e % 8, pl.ds(cs, 16)] = col

    def store(k, buf):
        st = staging[buf]
        pltpu.async_copy(st.at[0], out_hbm.at[k, 0, pl.ds(wid * BT_PER_W,
                                                          BT_PER_W)],
                         ssem[buf])
        pltpu.async_copy(st.at[1], out_hbm.at[k, 1, pl.ds(wid * BT_PER_W,
                                                          BT_PER_W)],
                         ssem[buf])

    def store_wait(k, buf):
        st = staging[buf]
        for et in range(2):
            pltpu.make_async_copy(
                st.at[et],
                out_hbm.at[k, et, pl.ds(wid * BT_PER_W, BT_PER_W)],
                ssem[buf]).wait()

    compute(0, 0)
    store(0, 0)
    compute(1, 1)
    store(1, 1)

    @pl.loop(2, K, step=2)
    def _pair(k):
        store_wait(k - 2, 0)
        compute(k, 0)
        store(k, 0)
        store_wait(k - 1, 1)
        compute(k + 1, 1)
        store(k + 1, 1)

    store_wait(K - 2, 0)
    store_wait(K - 1, 1)


_sc_gather = functools.partial(
    pl.kernel,
    out_type=jax.ShapeDtypeStruct((K, 2, 128, 8, 128), jnp.float32),
    mesh=plsc.VectorSubcoreMesh(core_axis_name="c", subcore_axis_name="s"),
    scratch_types=(
        [pltpu.VMEM((TPAD,), jnp.float32) for _ in range(EMBED_DIM)]
        + [
            pltpu.VMEM((K, B_PER_W), jnp.int32),
            pltpu.VMEM((2, BT_PER_W, 8, 128), jnp.float32),
            pltpu.VMEM((2, BT_PER_W, 8, 128), jnp.float32),
            pltpu.SemaphoreType.DMA,
            pltpu.SemaphoreType.DMA,
        ]
    ),
    compiler_params=pltpu.CompilerParams(use_tc_tiling_on_sc=False,
                                         needs_layout_passes=False),
)(_sc_gather_body)


@jax.jit
def kernel(type_indices, glove_embeddings, W, b):
    table_t = _project_table(glove_embeddings, W, b)
    idx_t = type_indices.T.astype(jnp.int32)
    out5 = _sc_gather(table_t, idx_t)
    return out5.transpose(2, 4, 0, 1, 3).reshape(B, K, EMBED_DIM)
